# baseline (speedup 1.0000x reference)
import functools

import jax
import jax.numpy as jnp
from jax import lax
from jax.experimental import pallas as pl
from jax.experimental.pallas import tpu as pltpu
from jax.experimental.pallas import tpu_sc as plsc

NUM_EXPERTS = 8
TOP_K = 2
HIDDEN = 768
INTER = 3072
TOKENS = 2048

BM = 256
PADDED = TOP_K * TOKENS + NUM_EXPERTS * BM
NBLK = PADDED // BM
KSPLIT = 2
IB = INTER // KSPLIT


def _mlp_block_kernel(be_ref, xs_ref, w1_ref, w3_ref, w2_ref, ws_ref, o_ref):
    k = pl.program_id(1)
    x = xs_ref[...]
    w1b = w1_ref[0]
    w3b = w3_ref[0]
    w2b = w2_ref[0]
    gate = jax.lax.dot_general(x, w1b, (((1,), (1,)), ((), ())),
                               preferred_element_type=jnp.float32)
    up = jax.lax.dot_general(x, w3b, (((1,), (1,)), ((), ())),
                             preferred_element_type=jnp.float32)
    h = gate * jax.nn.sigmoid(gate) * up
    o = jax.lax.dot_general(h, w2b, (((1,), (1,)), ((), ())),
                            preferred_element_type=jnp.float32)
    o = o * ws_ref[0, 0][:, None]

    @pl.when(k == 0)
    def _():
        o_ref[...] = o

    @pl.when(k != 0)
    def _():
        o_ref[...] += o


def _grouped_mlp(xs, w1, w3, w2, ws3d, block_expert):
    def kk(i, k):
        return jax.lax.bitwise_xor(k, i % 2)

    grid_spec = pltpu.PrefetchScalarGridSpec(
        num_scalar_prefetch=1,
        grid=(NBLK, KSPLIT),
        in_specs=[
            pl.BlockSpec((BM, HIDDEN), lambda i, k, be: (i, 0)),
            pl.BlockSpec((1, IB, HIDDEN), lambda i, k, be: (be[i], kk(i, k), 0)),
            pl.BlockSpec((1, IB, HIDDEN), lambda i, k, be: (be[i], kk(i, k), 0)),
            pl.BlockSpec((1, HIDDEN, IB), lambda i, k, be: (be[i], 0, kk(i, k))),
            pl.BlockSpec((1, 1, BM), lambda i, k, be: (i, 0, 0)),
        ],
        out_specs=pl.BlockSpec((BM, HIDDEN), lambda i, k, be: (i, 0)),
    )
    return pl.pallas_call(
        _mlp_block_kernel,
        grid_spec=grid_spec,
        out_shape=jax.ShapeDtypeStruct((PADDED, HIDDEN), jnp.float32),
        compiler_params=pltpu.CompilerParams(
            dimension_semantics=("arbitrary", "arbitrary")),
    )(block_expert, xs, w1, w3, w2, ws3d)


_NC, _NS = 2, 16
_NW = _NC * _NS
_SLOTS_PER_W = PADDED // _NW
_GCHUNK = 64
_TOK_PER_W = TOKENS // _NW
_CCHUNK = 16
_POSPAD = 2 * TOKENS + 128


def _sc_mesh():
    return plsc.VectorSubcoreMesh(core_axis_name="c", subcore_axis_name="s")


def _gather_rows(x, idx, n_rows):
    per_w = n_rows // _NW
    nch = per_w // _GCHUNK

    @functools.partial(
        pl.kernel,
        out_type=jax.ShapeDtypeStruct((n_rows, HIDDEN), jnp.float32),
        mesh=_sc_mesh(),
        scratch_types=[
            pltpu.VMEM((2, _GCHUNK), jnp.int32),
            pltpu.VMEM((_GCHUNK, HIDDEN), jnp.float32),
            pltpu.VMEM((_GCHUNK, HIDDEN), jnp.float32),
            pltpu.SemaphoreType.DMA,
            pltpu.SemaphoreType.DMA,
            pltpu.SemaphoreType.DMA,
            pltpu.SemaphoreType.DMA,
        ],
    )
    def k(x_hbm, tok_hbm, xs_hbm, idx_v, rows0, rows1, g0, g1, w0, w1):
        wid = lax.axis_index("s") * _NC + lax.axis_index("c")
        base = wid * per_w
        rows = (rows0, rows1)
        gsem = (g0, g1)
        wsem = (w0, w1)
        gathers = [None] * nch
        writes = [None] * nch
        for c in range(nch):
            off = base + c * _GCHUNK
            pltpu.sync_copy(tok_hbm.at[pl.ds(off, _GCHUNK)], idx_v.at[c % 2])
            if c >= 2:
                writes[c - 2].wait()
            gathers[c] = pltpu.async_copy(
                x_hbm.at[idx_v.at[c % 2]], rows[c % 2], gsem[c % 2])
            gathers[c].wait()
            writes[c] = pltpu.async_copy(
                rows[c % 2], xs_hbm.at[pl.ds(off, _GCHUNK)], wsem[c % 2])
        for c in range(max(0, nch - 2), nch):
            writes[c].wait()

    return k(x, idx)


def _pair_add_kernel(a_ref, b_ref, o_ref):
    o_ref[...] = a_ref[...] + b_ref[...]


def _combine_rows(o_sorted, pos):
    pairs = _gather_rows(o_sorted, pos, 2 * TOKENS)
    badd = 512
    nb = TOKENS // badd
    return pl.pallas_call(
        _pair_add_kernel,
        grid=(nb,),
        in_specs=[
            pl.BlockSpec((badd, HIDDEN), lambda i: (i, 0)),
            pl.BlockSpec((badd, HIDDEN), lambda i: (i + nb, 0)),
        ],
        out_specs=pl.BlockSpec((badd, HIDDEN), lambda i: (i, 0)),
        out_shape=jax.ShapeDtypeStruct((TOKENS, HIDDEN), jnp.float32),
    )(pairs, pairs)


def kernel(hidden_states, router_logits, w1, w3, w2):
    x = hidden_states.reshape(-1, HIDDEN)

    topv, topi = jax.lax.top_k(router_logits, TOP_K)
    rw = jax.nn.softmax(topv, axis=-1)
    e_flat = topi.reshape(-1)
    t_flat = jnp.repeat(jnp.arange(TOKENS, dtype=jnp.int32), TOP_K)
    w_flat = rw.reshape(-1)

    counts = jnp.bincount(e_flat, length=NUM_EXPERTS)
    aligned = ((counts + BM - 1) // BM) * BM
    a_off = jnp.concatenate([jnp.zeros((1,), jnp.int32),
                             jnp.cumsum(aligned)[:-1].astype(jnp.int32)])
    order = jnp.argsort(e_flat, stable=True)
    e_sorted = e_flat[order]
    grp_start = a_off[e_sorted]
    c_off = jnp.concatenate([jnp.zeros((1,), jnp.int32),
                             jnp.cumsum(counts)[:-1].astype(jnp.int32)])
    pos_in_grp = jnp.arange(2 * TOKENS, dtype=jnp.int32) - c_off[e_sorted]
    slot = grp_start + pos_in_grp

    tok_sorted = jnp.zeros((PADDED,), jnp.int32).at[slot].set(
        t_flat[order].astype(jnp.int32))
    ws_buf = jnp.zeros((PADDED,), jnp.float32).at[slot].set(w_flat[order])
    pos_i = jnp.zeros((_POSPAD,), jnp.int32).at[order].set(slot)[:2 * TOKENS]
    pos = jnp.concatenate([pos_i[0::2], pos_i[1::2]])

    blk_off = (a_off // BM).astype(jnp.int32)
    block_expert = jnp.clip(
        (jnp.arange(NBLK, dtype=jnp.int32)[:, None] >= blk_off[None, :])
        .sum(axis=1) - 1, 0, NUM_EXPERTS - 1).astype(jnp.int32)

    xs = _gather_rows(x, tok_sorted, PADDED)

    ws3d = ws_buf.reshape(NBLK, 1, BM)
    o_sorted = _grouped_mlp(xs, w1, w3, w2, ws3d, block_expert)

    out = _combine_rows(o_sorted, pos)
    return out


# speedup vs baseline: 1.2606x; 1.2606x over previous
import functools

import jax
import jax.numpy as jnp
from jax import lax
from jax.experimental import pallas as pl
from jax.experimental.pallas import tpu as pltpu
from jax.experimental.pallas import tpu_sc as plsc

NUM_EXPERTS = 8
TOP_K = 2
HIDDEN = 768
INTER = 3072
TOKENS = 2048

BM = 256
PADDED = TOP_K * TOKENS + NUM_EXPERTS * BM
NBLK = PADDED // BM
KSPLIT = 2
IB = INTER // KSPLIT


def _mlp_block_kernel(be_ref, xs_ref, w1_ref, w3_ref, w2_ref, ws_ref, o_ref):
    k = pl.program_id(1)
    x = xs_ref[...]
    w1b = w1_ref[0]
    w3b = w3_ref[0]
    w2b = w2_ref[0]
    gate = jax.lax.dot_general(x, w1b, (((1,), (1,)), ((), ())),
                               preferred_element_type=jnp.float32)
    up = jax.lax.dot_general(x, w3b, (((1,), (1,)), ((), ())),
                             preferred_element_type=jnp.float32)
    h = gate * jax.nn.sigmoid(gate) * up
    o = jax.lax.dot_general(h, w2b, (((1,), (1,)), ((), ())),
                            preferred_element_type=jnp.float32)
    o = o * ws_ref[0, 0][:, None]

    @pl.when(k == 0)
    def _():
        o_ref[...] = o

    @pl.when(k != 0)
    def _():
        o_ref[...] += o


def _grouped_mlp(xs, w1, w3, w2, ws3d, block_expert):
    def kk(i, k):
        return jax.lax.bitwise_xor(k, i % 2)

    grid_spec = pltpu.PrefetchScalarGridSpec(
        num_scalar_prefetch=1,
        grid=(NBLK, KSPLIT),
        in_specs=[
            pl.BlockSpec((BM, HIDDEN), lambda i, k, be: (i, 0)),
            pl.BlockSpec((1, IB, HIDDEN), lambda i, k, be: (be[i], kk(i, k), 0)),
            pl.BlockSpec((1, IB, HIDDEN), lambda i, k, be: (be[i], kk(i, k), 0)),
            pl.BlockSpec((1, HIDDEN, IB), lambda i, k, be: (be[i], 0, kk(i, k))),
            pl.BlockSpec((1, 1, BM), lambda i, k, be: (i, 0, 0)),
        ],
        out_specs=pl.BlockSpec((BM, HIDDEN), lambda i, k, be: (i, 0)),
    )
    return pl.pallas_call(
        _mlp_block_kernel,
        grid_spec=grid_spec,
        out_shape=jax.ShapeDtypeStruct((PADDED, HIDDEN), jnp.float32),
        compiler_params=pltpu.CompilerParams(
            dimension_semantics=("arbitrary", "arbitrary")),
    )(block_expert, xs, w1, w3, w2, ws3d)


_NC, _NS = 2, 16
_NW = _NC * _NS
_SLOTS_PER_W = PADDED // _NW
_GCHUNK = 64
_TOK_PER_W = TOKENS // _NW
_CCHUNK = 16
_POSPAD = 2 * TOKENS + 128


def _sc_mesh():
    return plsc.VectorSubcoreMesh(core_axis_name="c", subcore_axis_name="s")


def _gather_rows(x, idx, n_rows):
    per_w = n_rows // _NW
    nch = per_w // _GCHUNK

    @functools.partial(
        pl.kernel,
        out_type=jax.ShapeDtypeStruct((n_rows, HIDDEN), jnp.float32),
        mesh=_sc_mesh(),
        scratch_types=[
            pltpu.VMEM((2, _GCHUNK), jnp.int32),
            pltpu.VMEM((_GCHUNK, HIDDEN), jnp.float32),
            pltpu.VMEM((_GCHUNK, HIDDEN), jnp.float32),
            pltpu.SemaphoreType.DMA,
            pltpu.SemaphoreType.DMA,
            pltpu.SemaphoreType.DMA,
            pltpu.SemaphoreType.DMA,
        ],
    )
    def k(x_hbm, tok_hbm, xs_hbm, idx_v, rows0, rows1, g0, g1, w0, w1):
        wid = lax.axis_index("s") * _NC + lax.axis_index("c")
        base = wid * per_w
        rows = (rows0, rows1)
        gsem = (g0, g1)
        wsem = (w0, w1)
        gathers = [None] * nch
        writes = [None] * nch
        for c in range(nch):
            off = base + c * _GCHUNK
            pltpu.sync_copy(tok_hbm.at[pl.ds(off, _GCHUNK)], idx_v.at[c % 2])
            if c >= 2:
                writes[c - 2].wait()
            gathers[c] = pltpu.async_copy(
                x_hbm.at[idx_v.at[c % 2]], rows[c % 2], gsem[c % 2])
            gathers[c].wait()
            writes[c] = pltpu.async_copy(
                rows[c % 2], xs_hbm.at[pl.ds(off, _GCHUNK)], wsem[c % 2])
        for c in range(max(0, nch - 2), nch):
            writes[c].wait()

    return k(x, idx)


def _pair_add_kernel(a_ref, b_ref, o_ref):
    o_ref[...] = a_ref[...] + b_ref[...]


def _combine_rows(o_sorted, pos):
    pairs = _gather_rows(o_sorted, pos, 2 * TOKENS)
    badd = 512
    nb = TOKENS // badd
    return pl.pallas_call(
        _pair_add_kernel,
        grid=(nb,),
        in_specs=[
            pl.BlockSpec((badd, HIDDEN), lambda i: (i, 0)),
            pl.BlockSpec((badd, HIDDEN), lambda i: (i + nb, 0)),
        ],
        out_specs=pl.BlockSpec((badd, HIDDEN), lambda i: (i, 0)),
        out_shape=jax.ShapeDtypeStruct((TOKENS, HIDDEN), jnp.float32),
    )(pairs, pairs)


def kernel(hidden_states, router_logits, w1, w3, w2):
    x = hidden_states.reshape(-1, HIDDEN)

    topv, topi = jax.lax.top_k(router_logits, TOP_K)
    rw = jax.nn.softmax(topv, axis=-1)
    e_flat = topi.reshape(-1)
    t_flat = jnp.repeat(jnp.arange(TOKENS, dtype=jnp.int32), TOP_K)
    w_flat = rw.reshape(-1)

    counts = jnp.bincount(e_flat, length=NUM_EXPERTS)
    aligned = ((counts + BM - 1) // BM) * BM
    a_off = jnp.concatenate([jnp.zeros((1,), jnp.int32),
                             jnp.cumsum(aligned)[:-1].astype(jnp.int32)])
    order = jnp.argsort(e_flat, stable=True)
    e_sorted = e_flat[order]
    grp_start = a_off[e_sorted]
    c_off = jnp.concatenate([jnp.zeros((1,), jnp.int32),
                             jnp.cumsum(counts)[:-1].astype(jnp.int32)])
    pos_in_grp = jnp.arange(2 * TOKENS, dtype=jnp.int32) - c_off[e_sorted]
    slot = grp_start + pos_in_grp

    pad_idx = (jnp.arange(PADDED, dtype=jnp.int32) * 997) % TOKENS
    tok_sorted = pad_idx.at[slot].set(t_flat[order].astype(jnp.int32))
    ws_buf = jnp.zeros((PADDED,), jnp.float32).at[slot].set(w_flat[order])
    pos_i = jnp.zeros((_POSPAD,), jnp.int32).at[order].set(slot)[:2 * TOKENS]
    pos = jnp.concatenate([pos_i[0::2], pos_i[1::2]])

    blk_off = (a_off // BM).astype(jnp.int32)
    block_expert = jnp.clip(
        (jnp.arange(NBLK, dtype=jnp.int32)[:, None] >= blk_off[None, :])
        .sum(axis=1) - 1, 0, NUM_EXPERTS - 1).astype(jnp.int32)

    xs = _gather_rows(x, tok_sorted, PADDED)

    ws3d = ws_buf.reshape(NBLK, 1, BM)
    o_sorted = _grouped_mlp(xs, w1, w3, w2, ws3d, block_expert)

    out = _combine_rows(o_sorted, pos)
    return out
